# baseline (speedup 1.0000x reference)
import functools

import jax
import jax.numpy as jnp
from jax import lax
from jax.experimental import pallas as pl
from jax.experimental.pallas import tpu as pltpu
from jax.experimental.pallas import tpu_sc as plsc

B, S, D = 16, 512, 512
N = 16384
H = 64
K = 64
SPW = 0.125
HID = 100
TSE_W = 2 * HID
TROLE_W = 4 * HID
HALF = N // 2
CHUNK = 2048
G = 8
CAND = 80


def _k1_body(seq_ref, wp_ref, bp_ref, wse_ref, bse_ref, wro_ref, bro_ref,
             tse_ref, tro_ref, h_ref):
    x = jnp.dot(seq_ref[0], wp_ref[...], preferred_element_type=jnp.float32)
    h = jnp.tanh(x + bp_ref[...])
    h_ref[0] = h
    tse_ref[0] = jnp.dot(h, wse_ref[...], preferred_element_type=jnp.float32) + bse_ref[...]
    tro_ref[0] = jnp.dot(h, wro_ref[...], preferred_element_type=jnp.float32) + bro_ref[...]


def _k1(seq, wp, bp, wse, bse, wro, bro):
    return pl.pallas_call(
        _k1_body,
        grid=(B,),
        in_specs=[
            pl.BlockSpec((1, S, D), lambda b: (b, 0, 0)),
            pl.BlockSpec((D, H), lambda b: (0, 0)),
            pl.BlockSpec((1, H), lambda b: (0, 0)),
            pl.BlockSpec((H, TSE_W), lambda b: (0, 0)),
            pl.BlockSpec((1, TSE_W), lambda b: (0, 0)),
            pl.BlockSpec((H, TROLE_W), lambda b: (0, 0)),
            pl.BlockSpec((1, TROLE_W), lambda b: (0, 0)),
        ],
        out_specs=[
            pl.BlockSpec((1, S, TSE_W), lambda b: (b, 0, 0)),
            pl.BlockSpec((1, S, TROLE_W), lambda b: (b, 0, 0)),
            pl.BlockSpec((1, S, H), lambda b: (b, 0, 0)),
        ],
        out_shape=[
            jax.ShapeDtypeStruct((B, S, TSE_W), jnp.float32),
            jax.ShapeDtypeStruct((B, S, TROLE_W), jnp.float32),
            jax.ShapeDtypeStruct((B, S, H), jnp.float32),
        ],
    )(seq, wp, bp, wse, bse, wro, bro)


@functools.cache
def _build_k2():
  mesh = plsc.VectorSubcoreMesh(core_axis_name="c", subcore_axis_name="s")
  deco = functools.partial(
    pl.kernel,
    out_type=[
        jax.ShapeDtypeStruct((B, N), jnp.float32),
        jax.ShapeDtypeStruct((B, N), jnp.float32),
        jax.ShapeDtypeStruct((B, N), jnp.float32),
        jax.ShapeDtypeStruct((B, N), jnp.float32),
        jax.ShapeDtypeStruct((B, 2, 16), jnp.float32),
    ],
    mesh=mesh,
    scratch_types=[
        pltpu.VMEM((S * TSE_W,), jnp.float32),
        pltpu.VMEM((CHUNK,), jnp.int32),
        pltpu.VMEM((CHUNK,), jnp.int32),
        pltpu.VMEM((CHUNK,), jnp.float32),
        pltpu.VMEM((CHUNK,), jnp.float32),
        pltpu.VMEM((CHUNK,), jnp.float32),
        pltpu.VMEM((CHUNK,), jnp.float32),
        pltpu.VMEM((CHUNK,), jnp.float32),
        pltpu.VMEM((416,), jnp.float32),
        pltpu.VMEM((16,), jnp.float32),
    ],
    compiler_params=pltpu.CompilerParams(needs_layout_passes=False),
  )

  @deco
  def k2(tse_hbm, st_hbm, en_hbm, mk_hbm, w2_hbm,
        s0_hbm, s1_hbm, s2_hbm, pr_hbm, sq_hbm,
        tab_v, st_v, en_v, mk_v, o0_v, o1_v, o2_v, opr_v, w2_v, sq_v):
    wid = lax.axis_index("s") * 2 + lax.axis_index("c")
    b = wid // 2
    half = wid % 2

    pltpu.sync_copy(tse_hbm.at[b], tab_v)
    pltpu.sync_copy(w2_hbm, w2_v)

    zi = jnp.zeros((16,), jnp.int32)
    b2s = [plsc.load_gather(w2_v, [zi + (384 + l)]) for l in range(3)]

    def chunk_step(ci, seqmax):
        base = half * HALF + ci * CHUNK
        pltpu.sync_copy(st_hbm.at[b, pl.ds(base, CHUNK)], st_v)
        pltpu.sync_copy(en_hbm.at[b, pl.ds(base, CHUNK)], en_v)
        pltpu.sync_copy(mk_hbm.at[b, pl.ds(base, CHUNK)], mk_v)

        def gblock_step(gb, seqmax):
            g0 = gb * (G * 16)
            svs = [st_v[pl.ds(g0 + g * 16, 16)] for g in range(G)]
            evs = [en_v[pl.ds(g0 + g * 16, 16)] for g in range(G)]
            fs0 = tuple(sv * TSE_W for sv in svs)
            fe0 = tuple(ev * TSE_W + HID for ev in evs)
            za = tuple(jnp.zeros((16,), jnp.float32) for _ in range(3 * G))

            def h_step(h, carry):
                accs = list(carry[0])
                fss = list(carry[1])
                fes = list(carry[2])
                w2s = [plsc.load_gather(w2_v, [zi + (l * 128) + h])
                       for l in range(3)]
                for g in range(G):
                    ts = plsc.load_gather(tab_v, [fss[g]])
                    te = plsc.load_gather(tab_v, [fes[g]])
                    gv = jnp.maximum(ts + te, 0.0)
                    accs[3 * g + 0] = accs[3 * g + 0] + gv * w2s[0]
                    accs[3 * g + 1] = accs[3 * g + 1] + gv * w2s[1]
                    accs[3 * g + 2] = accs[3 * g + 2] + gv * w2s[2]
                    fss[g] = fss[g] + 1
                    fes[g] = fes[g] + 1
                return (tuple(accs), tuple(fss), tuple(fes))

            accs, _, _ = lax.fori_loop(0, HID, h_step, (za, fs0, fe0))
            for g in range(G):
                off = g0 + g * 16
                a0 = accs[3 * g + 0] + b2s[0]
                a1 = accs[3 * g + 1] + b2s[1]
                a2 = accs[3 * g + 2] + b2s[2]
                mkv = mk_v[pl.ds(off, 16)]
                prune = jnp.where(mkv > 0.0, jnp.maximum(a1, a2), -1e30)
                o0_v[pl.ds(off, 16)] = a0
                o1_v[pl.ds(off, 16)] = a1
                o2_v[pl.ds(off, 16)] = a2
                opr_v[pl.ds(off, 16)] = prune
                ef = evs[g].astype(jnp.float32)
                seqmax = jnp.maximum(seqmax, mkv * ef)
            return seqmax

        seqmax = lax.fori_loop(0, CHUNK // (G * 16), gblock_step, seqmax)
        pltpu.sync_copy(o0_v, s0_hbm.at[b, pl.ds(base, CHUNK)])
        pltpu.sync_copy(o1_v, s1_hbm.at[b, pl.ds(base, CHUNK)])
        pltpu.sync_copy(o2_v, s2_hbm.at[b, pl.ds(base, CHUNK)])
        pltpu.sync_copy(opr_v, pr_hbm.at[b, pl.ds(base, CHUNK)])
        return seqmax

    seqmax = lax.fori_loop(0, HALF // CHUNK, chunk_step,
                           jnp.zeros((16,), jnp.float32))
    sq_v[...] = seqmax
    pltpu.sync_copy(sq_v, sq_hbm.at[b, half])

  return k2


def _k3_body(prm_ref, pk_ref, cp_ref, ci_ref):
    x = prm_ref[0]
    pk = pk_ref[0]
    fio = (lax.broadcasted_iota(jnp.int32, (128, 128), 0) * 128
           + lax.broadcasted_iota(jnp.int32, (128, 128), 1))
    ic = lax.broadcasted_iota(jnp.int32, (1, CAND), 1)
    bigi = jnp.int32(2147483647)

    def step(k, carry):
        x, cp, ci = carry
        m = jnp.max(x)
        flat = jnp.min(jnp.where(x == m, fio, bigi))
        hit1 = fio == flat
        pv = jnp.max(jnp.where(hit1, pk, 0))
        sel = ic == k
        cp = jnp.where(sel, pv, cp)
        ci = jnp.where(sel, flat, ci)
        x = jnp.where(hit1, -3e38, x)
        return x, cp, ci

    zi = jnp.zeros((1, CAND), jnp.int32)
    x, cp, ci = lax.fori_loop(0, CAND, step, (x, zi, zi))
    cp_ref[0] = cp
    ci_ref[0] = ci


def _k3(prm, pk):
    return pl.pallas_call(
        _k3_body,
        grid=(B,),
        in_specs=[
            pl.BlockSpec((1, 128, 128), lambda b: (b, 0, 0)),
            pl.BlockSpec((1, 128, 128), lambda b: (b, 0, 0)),
        ],
        out_specs=[
            pl.BlockSpec((1, 1, CAND), lambda b: (b, 0, 0)),
            pl.BlockSpec((1, 1, CAND), lambda b: (b, 0, 0)),
        ],
        out_shape=[
            jax.ShapeDtypeStruct((B, 1, CAND), jnp.int32),
            jax.ShapeDtypeStruct((B, 1, CAND), jnp.int32),
        ],
    )(prm, pk)


def _k4_body(cp_ref, ci_ref, h_ref, w1_ref, b1_ref, w2_ref,
             b2_ref, tro_ref, sq_ref, r2_ref, rb2_ref,
             role_ref, tops_ref, tope_ref, tsm_ref):
    cpv = cp_ref[0]
    csv = lax.shift_right_logical(cpv, 10)
    cev = lax.shift_right_logical(cpv, 1) & 511
    cmv = (cpv & 1).astype(jnp.float32)
    civ = ci_ref[0]
    i512c = lax.broadcasted_iota(jnp.int32, (CAND, S), 1)
    ohs = jnp.where(csv.reshape(CAND, 1) == i512c, 1.0, 0.0)
    ohe = jnp.where(cev.reshape(CAND, 1) == i512c, 1.0, 0.0)
    embs = jnp.dot(ohs, h_ref[0], preferred_element_type=jnp.float32,
                   precision=lax.Precision.HIGHEST)
    embe = jnp.dot(ohe, h_ref[0], preferred_element_type=jnp.float32,
                   precision=lax.Precision.HIGHEST)
    emb = jnp.concatenate([embs, embe], axis=1)
    u = jnp.dot(emb, w1_ref[...], preferred_element_type=jnp.float32) + b1_ref[...]
    g = jnp.maximum(u, 0.0)
    sc3 = jnp.dot(g, w2_ref[...], preferred_element_type=jnp.float32) + b2_ref[...]
    pr = jnp.max(sc3[:, 1:3], axis=1)
    pex = jnp.where(cmv > 0.0, pr.reshape(1, CAND), -1e30)

    i64 = lax.broadcasted_iota(jnp.int32, (1, K), 1)
    big = jnp.int32(2147483647)

    def step(k, carry):
        pex, tops, tope, topm = carry
        m = jnp.max(pex)
        jidx = jnp.min(jnp.where(pex == m, civ, big))
        hit = (pex == m) & (civ == jidx)
        sel = i64 == k
        tops = jnp.where(sel, jnp.sum(jnp.where(hit, csv, 0)), tops)
        tope = jnp.where(sel, jnp.sum(jnp.where(hit, cev, 0)), tope)
        topm = jnp.where(sel, jnp.sum(jnp.where(hit, cmv, 0.0)), topm)
        pex = jnp.where(hit, -3e38, pex)
        return pex, tops, tope, topm

    zi = jnp.zeros((1, K), jnp.int32)
    _, tops, tope, topm = lax.fori_loop(
        0, K, step, (pex, zi, zi, jnp.zeros((1, K), jnp.float32)))

    sl = jnp.max(sq_ref[...])
    nk = jnp.clip(jnp.ceil(SPW * sl), 1.0, float(K))
    tsm = jnp.where(i64.astype(jnp.float32) < nk, topm, 0.0)

    i512 = lax.broadcasted_iota(jnp.int32, (K, S), 1)
    ohs2 = jnp.where(tops.reshape(K, 1) == i512, 1.0, 0.0)
    ohe2 = jnp.where(tope.reshape(K, 1) == i512, 1.0, 0.0)
    gs = jnp.dot(ohs2, tro_ref[0], preferred_element_type=jnp.float32)
    ge = jnp.dot(ohe2, tro_ref[0], preferred_element_type=jnp.float32)
    p = gs[:, 0:HID] + ge[:, HID:2 * HID]
    q = gs[:, 2 * HID:3 * HID] + ge[:, 3 * HID:4 * HID]
    hid = jnp.maximum(
        jnp.broadcast_to(p.reshape(K, 1, HID), (K, K, HID))
        + jnp.broadcast_to(q.reshape(1, K, HID), (K, K, HID)), 0.0
    ).reshape(K * K, HID)
    role_ref[0] = jnp.dot(hid, r2_ref[...], preferred_element_type=jnp.float32) + rb2_ref[...]
    tops_ref[0] = tops
    tope_ref[0] = tope
    tsm_ref[0] = tsm


def _k4(cp, ci, h, w1, b1, w2, b2, tro, sq, r2, rb2):
    return pl.pallas_call(
        _k4_body,
        grid=(B,),
        in_specs=[
            pl.BlockSpec((1, 1, CAND), lambda b: (b, 0, 0)),
            pl.BlockSpec((1, 1, CAND), lambda b: (b, 0, 0)),
            pl.BlockSpec((1, S, H), lambda b: (b, 0, 0)),
            pl.BlockSpec((2 * H, HID), lambda b: (0, 0)),
            pl.BlockSpec((1, HID), lambda b: (0, 0)),
            pl.BlockSpec((HID, 3), lambda b: (0, 0)),
            pl.BlockSpec((1, 3), lambda b: (0, 0)),
            pl.BlockSpec((1, S, TROLE_W), lambda b: (b, 0, 0)),
            pl.BlockSpec((1, 1, 32), lambda b: (b, 0, 0)),
            pl.BlockSpec((HID, 2), lambda b: (0, 0)),
            pl.BlockSpec((1, 2), lambda b: (0, 0)),
        ],
        out_specs=[
            pl.BlockSpec((1, K * K, 2), lambda b: (b, 0, 0)),
            pl.BlockSpec((1, 1, K), lambda b: (b, 0, 0)),
            pl.BlockSpec((1, 1, K), lambda b: (b, 0, 0)),
            pl.BlockSpec((1, 1, K), lambda b: (b, 0, 0)),
        ],
        out_shape=[
            jax.ShapeDtypeStruct((B, K * K, 2), jnp.float32),
            jax.ShapeDtypeStruct((B, 1, K), jnp.int32),
            jax.ShapeDtypeStruct((B, 1, K), jnp.int32),
            jax.ShapeDtypeStruct((B, 1, K), jnp.float32),
        ],
    )(cp, ci, h, w1, b1, w2, b2, tro, sq, r2, rb2)


def kernel(seq_tensor, span_indices, span_mask, W_proj, b_proj, W1, b1, W2,
           b2, R1, rb1, R2, rb2):
    f32 = jnp.float32
    starts = span_indices[..., 0]
    ends = span_indices[..., 1]

    wse = jnp.concatenate([W1[:H], W1[H:]], axis=1)
    bse = jnp.concatenate([jnp.zeros((HID,), f32), b1])[None, :]
    wro = jnp.concatenate([R1[:H], R1[H:2 * H], R1[2 * H:3 * H], R1[3 * H:]],
                          axis=1)
    bro = jnp.concatenate([jnp.zeros((HID,), f32), rb1,
                           jnp.zeros((2 * HID,), f32)])[None, :]
    pad28 = jnp.zeros((28,), f32)
    w2cat = jnp.concatenate([W2[:, 0], pad28, W2[:, 1], pad28, W2[:, 2],
                             pad28, b2, jnp.zeros((29,), f32)])

    tse, tro, h3 = _k1(seq_tensor, W_proj, b_proj[None, :], wse, bse, wro, bro)

    s0, s1, s2, prm, seqpart = _build_k2()(
        tse.reshape(B, S * TSE_W), starts, ends, span_mask, w2cat)

    pk = (starts * 1024 + ends * 2
          + (span_mask > 0).astype(jnp.int32)).reshape(B, 128, 128)
    cp, ci = _k3(prm.reshape(B, 128, 128), pk)

    role, tops, tope, tsm = _k4(
        cp, ci, h3, W1, b1[None, :], W2, b2[None, :], tro,
        seqpart.reshape(B, 1, 32), R2, rb2[None, :])

    span_scores = jnp.stack([s0, s1, s2], axis=-1)
    role_scores = role.reshape(B, K, K, 2)
    top_indices = jnp.stack([tops.reshape(B, K), tope.reshape(B, K)], axis=-1)
    return span_scores, role_scores, tsm.reshape(B, K), top_indices


# speedup vs baseline: 1.5548x; 1.5548x over previous
import functools

import jax
import jax.numpy as jnp
from jax import lax
from jax.experimental import pallas as pl
from jax.experimental.pallas import tpu as pltpu
from jax.experimental.pallas import tpu_sc as plsc

B, S, D = 16, 512, 512
N = 16384
H = 64
K = 64
SPW = 0.125
HID = 100
TSE_W = 2 * HID
TROLE_W = 4 * HID
HALF = N // 2
CHUNK = 2048
G = 8
CAND = 80


def _k1_body(seq_ref, wp_ref, bp_ref, wse_ref, bse_ref, wro_ref, bro_ref,
             tse_ref, tro_ref, h_ref):
    x = jnp.dot(seq_ref[0], wp_ref[...], preferred_element_type=jnp.float32)
    h = jnp.tanh(x + bp_ref[...])
    h_ref[0] = h
    tse_ref[0] = jnp.dot(h, wse_ref[...], preferred_element_type=jnp.float32) + bse_ref[...]
    tro_ref[0] = jnp.dot(h, wro_ref[...], preferred_element_type=jnp.float32) + bro_ref[...]


def _k1(seq, wp, bp, wse, bse, wro, bro):
    return pl.pallas_call(
        _k1_body,
        grid=(B,),
        in_specs=[
            pl.BlockSpec((1, S, D), lambda b: (b, 0, 0)),
            pl.BlockSpec((D, H), lambda b: (0, 0)),
            pl.BlockSpec((1, H), lambda b: (0, 0)),
            pl.BlockSpec((H, TSE_W), lambda b: (0, 0)),
            pl.BlockSpec((1, TSE_W), lambda b: (0, 0)),
            pl.BlockSpec((H, TROLE_W), lambda b: (0, 0)),
            pl.BlockSpec((1, TROLE_W), lambda b: (0, 0)),
        ],
        out_specs=[
            pl.BlockSpec((1, S, TSE_W), lambda b: (b, 0, 0)),
            pl.BlockSpec((1, S, TROLE_W), lambda b: (b, 0, 0)),
            pl.BlockSpec((1, S, H), lambda b: (b, 0, 0)),
        ],
        out_shape=[
            jax.ShapeDtypeStruct((B, S, TSE_W), jnp.float32),
            jax.ShapeDtypeStruct((B, S, TROLE_W), jnp.float32),
            jax.ShapeDtypeStruct((B, S, H), jnp.float32),
        ],
    )(seq, wp, bp, wse, bse, wro, bro)


@functools.cache
def _build_k2():
  mesh = plsc.VectorSubcoreMesh(core_axis_name="c", subcore_axis_name="s")
  deco = functools.partial(
    pl.kernel,
    out_type=[
        jax.ShapeDtypeStruct((B, N), jnp.float32),
        jax.ShapeDtypeStruct((B, N), jnp.float32),
        jax.ShapeDtypeStruct((B, N), jnp.float32),
        jax.ShapeDtypeStruct((B, N), jnp.float32),
        jax.ShapeDtypeStruct((B, 2, 16), jnp.float32),
    ],
    mesh=mesh,
    scratch_types=[
        pltpu.VMEM((S * TSE_W,), jnp.float32),
        pltpu.VMEM((CHUNK,), jnp.int32),
        pltpu.VMEM((CHUNK,), jnp.int32),
        pltpu.VMEM((CHUNK,), jnp.float32),
        pltpu.VMEM((CHUNK,), jnp.float32),
        pltpu.VMEM((CHUNK,), jnp.float32),
        pltpu.VMEM((CHUNK,), jnp.float32),
        pltpu.VMEM((CHUNK,), jnp.float32),
        pltpu.VMEM((416,), jnp.float32),
        pltpu.VMEM((16,), jnp.float32),
    ],
    compiler_params=pltpu.CompilerParams(needs_layout_passes=False),
  )

  @deco
  def k2(tse_hbm, st_hbm, en_hbm, mk_hbm, w2_hbm,
        s0_hbm, s1_hbm, s2_hbm, pr_hbm, sq_hbm,
        tab_v, st_v, en_v, mk_v, o0_v, o1_v, o2_v, opr_v, w2_v, sq_v):
    wid = lax.axis_index("s") * 2 + lax.axis_index("c")
    b = wid // 2
    half = wid % 2

    pltpu.sync_copy(tse_hbm.at[b], tab_v)
    pltpu.sync_copy(w2_hbm, w2_v)

    zi = jnp.zeros((16,), jnp.int32)
    b2s = [plsc.load_gather(w2_v, [zi + (384 + l)]) for l in range(3)]

    def chunk_step(ci, seqmax):
        base = half * HALF + ci * CHUNK
        pltpu.sync_copy(st_hbm.at[b, pl.ds(base, CHUNK)], st_v)
        pltpu.sync_copy(en_hbm.at[b, pl.ds(base, CHUNK)], en_v)
        pltpu.sync_copy(mk_hbm.at[b, pl.ds(base, CHUNK)], mk_v)

        def gblock_step(gb, seqmax):
            g0 = gb * (G * 16)
            svs = [st_v[pl.ds(g0 + g * 16, 16)] for g in range(G)]
            evs = [en_v[pl.ds(g0 + g * 16, 16)] for g in range(G)]
            fs0 = tuple(sv * TSE_W for sv in svs)
            fe0 = tuple(ev * TSE_W + HID for ev in evs)
            za = tuple(jnp.zeros((16,), jnp.float32) for _ in range(3 * G))

            def h_step(h, carry):
                accs = list(carry[0])
                fss = list(carry[1])
                fes = list(carry[2])
                w2s = [plsc.load_gather(w2_v, [zi + (l * 128) + h])
                       for l in range(3)]
                for g in range(G):
                    ts = plsc.load_gather(tab_v, [fss[g]])
                    te = plsc.load_gather(tab_v, [fes[g]])
                    gv = jnp.maximum(ts + te, 0.0)
                    accs[3 * g + 0] = accs[3 * g + 0] + gv * w2s[0]
                    accs[3 * g + 1] = accs[3 * g + 1] + gv * w2s[1]
                    accs[3 * g + 2] = accs[3 * g + 2] + gv * w2s[2]
                    fss[g] = fss[g] + 1
                    fes[g] = fes[g] + 1
                return (tuple(accs), tuple(fss), tuple(fes))

            accs, _, _ = lax.fori_loop(0, HID, h_step, (za, fs0, fe0))
            for g in range(G):
                off = g0 + g * 16
                a0 = accs[3 * g + 0] + b2s[0]
                a1 = accs[3 * g + 1] + b2s[1]
                a2 = accs[3 * g + 2] + b2s[2]
                mkv = mk_v[pl.ds(off, 16)]
                prune = jnp.where(mkv > 0.0, jnp.maximum(a1, a2), -1e30)
                o0_v[pl.ds(off, 16)] = a0
                o1_v[pl.ds(off, 16)] = a1
                o2_v[pl.ds(off, 16)] = a2
                opr_v[pl.ds(off, 16)] = prune
                ef = evs[g].astype(jnp.float32)
                seqmax = jnp.maximum(seqmax, mkv * ef)
            return seqmax

        seqmax = lax.fori_loop(0, CHUNK // (G * 16), gblock_step, seqmax)
        pltpu.sync_copy(o0_v, s0_hbm.at[b, pl.ds(base, CHUNK)])
        pltpu.sync_copy(o1_v, s1_hbm.at[b, pl.ds(base, CHUNK)])
        pltpu.sync_copy(o2_v, s2_hbm.at[b, pl.ds(base, CHUNK)])
        pltpu.sync_copy(opr_v, pr_hbm.at[b, pl.ds(base, CHUNK)])
        return seqmax

    seqmax = lax.fori_loop(0, HALF // CHUNK, chunk_step,
                           jnp.zeros((16,), jnp.float32))
    sq_v[...] = seqmax
    pltpu.sync_copy(sq_v, sq_hbm.at[b, half])

  return k2


def _k3_body(prm_ref, pk_ref, cp_ref, ci_ref):
    x = prm_ref[...]
    pk = pk_ref[...]
    fio = (lax.broadcasted_iota(jnp.int32, (B, 128, 128), 1) * 128
           + lax.broadcasted_iota(jnp.int32, (B, 128, 128), 2))
    ic = lax.broadcasted_iota(jnp.int32, (B, CAND), 1)
    bigi = jnp.int32(2147483647)

    def step(k, carry):
        x, cp, ci = carry
        m = jnp.max(jnp.max(x, axis=2, keepdims=True), axis=1, keepdims=True)
        flat = jnp.min(jnp.min(jnp.where(x == m, fio, bigi), axis=2,
                               keepdims=True), axis=1, keepdims=True)
        hit = fio == flat
        pv = jnp.max(jnp.max(jnp.where(hit, pk, 0), axis=2, keepdims=True),
                     axis=1, keepdims=True)
        sel = ic == k
        cp = jnp.where(sel, jnp.broadcast_to(pv[:, :, 0], (B, CAND)), cp)
        ci = jnp.where(sel, jnp.broadcast_to(flat[:, :, 0], (B, CAND)), ci)
        x = jnp.where(hit, -3e38, x)
        return x, cp, ci

    zi = jnp.zeros((B, CAND), jnp.int32)
    x, cp, ci = lax.fori_loop(0, CAND, step, (x, zi, zi))
    cp_ref[...] = cp
    ci_ref[...] = ci


def _k3(prm, pk):
    return pl.pallas_call(
        _k3_body,
        in_specs=[
            pl.BlockSpec((B, 128, 128), lambda: (0, 0, 0)),
            pl.BlockSpec((B, 128, 128), lambda: (0, 0, 0)),
        ],
        out_specs=[
            pl.BlockSpec((B, CAND), lambda: (0, 0)),
            pl.BlockSpec((B, CAND), lambda: (0, 0)),
        ],
        out_shape=[
            jax.ShapeDtypeStruct((B, CAND), jnp.int32),
            jax.ShapeDtypeStruct((B, CAND), jnp.int32),
        ],
    )(prm, pk)


def _k4_body(cp_ref, ci_ref, h_ref, w1_ref, b1_ref, w2_ref,
             b2_ref, tro_ref, sq_ref, r2_ref, rb2_ref,
             role_ref, tops_ref, tope_ref, tsm_ref):
    cpv = cp_ref[0]
    csv = lax.shift_right_logical(cpv, 10)
    cev = lax.shift_right_logical(cpv, 1) & 511
    cmv = (cpv & 1).astype(jnp.float32)
    civ = ci_ref[0]
    i512c = lax.broadcasted_iota(jnp.int32, (CAND, S), 1)
    ohs = jnp.where(csv.reshape(CAND, 1) == i512c, 1.0, 0.0)
    ohe = jnp.where(cev.reshape(CAND, 1) == i512c, 1.0, 0.0)
    embs = jnp.dot(ohs, h_ref[0], preferred_element_type=jnp.float32,
                   precision=lax.Precision.HIGHEST)
    embe = jnp.dot(ohe, h_ref[0], preferred_element_type=jnp.float32,
                   precision=lax.Precision.HIGHEST)
    emb = jnp.concatenate([embs, embe], axis=1)
    u = jnp.dot(emb, w1_ref[...], preferred_element_type=jnp.float32) + b1_ref[...]
    g = jnp.maximum(u, 0.0)
    sc3 = jnp.dot(g, w2_ref[...], preferred_element_type=jnp.float32) + b2_ref[...]
    pr = jnp.max(sc3[:, 1:3], axis=1)
    pex = jnp.where(cmv > 0.0, pr.reshape(1, CAND), -1e30)

    i64 = lax.broadcasted_iota(jnp.int32, (1, K), 1)
    big = jnp.int32(2147483647)

    def step(k, carry):
        pex, tops, tope, topm = carry
        m = jnp.max(pex)
        jidx = jnp.min(jnp.where(pex == m, civ, big))
        hit = (pex == m) & (civ == jidx)
        sel = i64 == k
        tops = jnp.where(sel, jnp.sum(jnp.where(hit, csv, 0)), tops)
        tope = jnp.where(sel, jnp.sum(jnp.where(hit, cev, 0)), tope)
        topm = jnp.where(sel, jnp.sum(jnp.where(hit, cmv, 0.0)), topm)
        pex = jnp.where(hit, -3e38, pex)
        return pex, tops, tope, topm

    zi = jnp.zeros((1, K), jnp.int32)
    _, tops, tope, topm = lax.fori_loop(
        0, K, step, (pex, zi, zi, jnp.zeros((1, K), jnp.float32)))

    sl = jnp.max(sq_ref[...])
    nk = jnp.clip(jnp.ceil(SPW * sl), 1.0, float(K))
    tsm = jnp.where(i64.astype(jnp.float32) < nk, topm, 0.0)

    i512 = lax.broadcasted_iota(jnp.int32, (K, S), 1)
    ohs2 = jnp.where(tops.reshape(K, 1) == i512, 1.0, 0.0)
    ohe2 = jnp.where(tope.reshape(K, 1) == i512, 1.0, 0.0)
    gs = jnp.dot(ohs2, tro_ref[0], preferred_element_type=jnp.float32)
    ge = jnp.dot(ohe2, tro_ref[0], preferred_element_type=jnp.float32)
    p = gs[:, 0:HID] + ge[:, HID:2 * HID]
    q = gs[:, 2 * HID:3 * HID] + ge[:, 3 * HID:4 * HID]
    hid = jnp.maximum(
        jnp.broadcast_to(p.reshape(K, 1, HID), (K, K, HID))
        + jnp.broadcast_to(q.reshape(1, K, HID), (K, K, HID)), 0.0
    ).reshape(K * K, HID)
    role_ref[0] = jnp.dot(hid, r2_ref[...], preferred_element_type=jnp.float32) + rb2_ref[...]
    tops_ref[0] = tops
    tope_ref[0] = tope
    tsm_ref[0] = tsm


def _k4(cp, ci, h, w1, b1, w2, b2, tro, sq, r2, rb2):
    return pl.pallas_call(
        _k4_body,
        grid=(B,),
        in_specs=[
            pl.BlockSpec((1, 1, CAND), lambda b: (b, 0, 0)),
            pl.BlockSpec((1, 1, CAND), lambda b: (b, 0, 0)),
            pl.BlockSpec((1, S, H), lambda b: (b, 0, 0)),
            pl.BlockSpec((2 * H, HID), lambda b: (0, 0)),
            pl.BlockSpec((1, HID), lambda b: (0, 0)),
            pl.BlockSpec((HID, 3), lambda b: (0, 0)),
            pl.BlockSpec((1, 3), lambda b: (0, 0)),
            pl.BlockSpec((1, S, TROLE_W), lambda b: (b, 0, 0)),
            pl.BlockSpec((1, 1, 32), lambda b: (b, 0, 0)),
            pl.BlockSpec((HID, 2), lambda b: (0, 0)),
            pl.BlockSpec((1, 2), lambda b: (0, 0)),
        ],
        out_specs=[
            pl.BlockSpec((1, K * K, 2), lambda b: (b, 0, 0)),
            pl.BlockSpec((1, 1, K), lambda b: (b, 0, 0)),
            pl.BlockSpec((1, 1, K), lambda b: (b, 0, 0)),
            pl.BlockSpec((1, 1, K), lambda b: (b, 0, 0)),
        ],
        out_shape=[
            jax.ShapeDtypeStruct((B, K * K, 2), jnp.float32),
            jax.ShapeDtypeStruct((B, 1, K), jnp.int32),
            jax.ShapeDtypeStruct((B, 1, K), jnp.int32),
            jax.ShapeDtypeStruct((B, 1, K), jnp.float32),
        ],
    )(cp, ci, h, w1, b1, w2, b2, tro, sq, r2, rb2)


def kernel(seq_tensor, span_indices, span_mask, W_proj, b_proj, W1, b1, W2,
           b2, R1, rb1, R2, rb2):
    f32 = jnp.float32
    starts = span_indices[..., 0]
    ends = span_indices[..., 1]

    wse = jnp.concatenate([W1[:H], W1[H:]], axis=1)
    bse = jnp.concatenate([jnp.zeros((HID,), f32), b1])[None, :]
    wro = jnp.concatenate([R1[:H], R1[H:2 * H], R1[2 * H:3 * H], R1[3 * H:]],
                          axis=1)
    bro = jnp.concatenate([jnp.zeros((HID,), f32), rb1,
                           jnp.zeros((2 * HID,), f32)])[None, :]
    pad28 = jnp.zeros((28,), f32)
    w2cat = jnp.concatenate([W2[:, 0], pad28, W2[:, 1], pad28, W2[:, 2],
                             pad28, b2, jnp.zeros((29,), f32)])

    tse, tro, h3 = _k1(seq_tensor, W_proj, b_proj[None, :], wse, bse, wro, bro)

    s0, s1, s2, prm, seqpart = _build_k2()(
        tse.reshape(B, S * TSE_W), starts, ends, span_mask, w2cat)

    pk = (starts * 1024 + ends * 2
          + (span_mask > 0).astype(jnp.int32)).reshape(B, 128, 128)
    cp, ci = _k3(prm.reshape(B, 128, 128), pk)
    cp = cp.reshape(B, 1, CAND)
    ci = ci.reshape(B, 1, CAND)

    role, tops, tope, tsm = _k4(
        cp, ci, h3, W1, b1[None, :], W2, b2[None, :], tro,
        seqpart.reshape(B, 1, 32), R2, rb2[None, :])

    span_scores = jnp.stack([s0, s1, s2], axis=-1)
    role_scores = role.reshape(B, K, K, 2)
    top_indices = jnp.stack([tops.reshape(B, K), tope.reshape(B, K)], axis=-1)
    return span_scores, role_scores, tsm.reshape(B, K), top_indices


# speedup vs baseline: 2.8489x; 1.8324x over previous
import functools

import jax
import jax.numpy as jnp
from jax import lax
from jax.experimental import pallas as pl
from jax.experimental.pallas import tpu as pltpu
from jax.experimental.pallas import tpu_sc as plsc

B, S, D = 16, 512, 512
N = 16384
H = 64
K = 64
SPW = 0.125
HID = 100
TSE_W = 2 * HID
TROLE_W = 4 * HID
HALF = N // 2
CHUNK = 2048
G = 8
CAND = 80


def _k1_body(seq_ref, wp_ref, bp_ref, wse_ref, bse_ref, wro_ref, bro_ref,
             tse_ref, tro_ref, h_ref):
    x = jnp.dot(seq_ref[0], wp_ref[...], preferred_element_type=jnp.float32)
    h = jnp.tanh(x + bp_ref[...])
    h_ref[0] = h
    tse_ref[0] = jnp.dot(h, wse_ref[...], preferred_element_type=jnp.float32) + bse_ref[...]
    tro_ref[0] = jnp.dot(h, wro_ref[...], preferred_element_type=jnp.float32) + bro_ref[...]


def _k1(seq, wp, bp, wse, bse, wro, bro):
    return pl.pallas_call(
        _k1_body,
        grid=(B,),
        in_specs=[
            pl.BlockSpec((1, S, D), lambda b: (b, 0, 0)),
            pl.BlockSpec((D, H), lambda b: (0, 0)),
            pl.BlockSpec((1, H), lambda b: (0, 0)),
            pl.BlockSpec((H, TSE_W), lambda b: (0, 0)),
            pl.BlockSpec((1, TSE_W), lambda b: (0, 0)),
            pl.BlockSpec((H, TROLE_W), lambda b: (0, 0)),
            pl.BlockSpec((1, TROLE_W), lambda b: (0, 0)),
        ],
        out_specs=[
            pl.BlockSpec((1, S, TSE_W), lambda b: (b, 0, 0)),
            pl.BlockSpec((1, S, TROLE_W), lambda b: (b, 0, 0)),
            pl.BlockSpec((1, S, H), lambda b: (b, 0, 0)),
        ],
        out_shape=[
            jax.ShapeDtypeStruct((B, S, TSE_W), jnp.float32),
            jax.ShapeDtypeStruct((B, S, TROLE_W), jnp.float32),
            jax.ShapeDtypeStruct((B, S, H), jnp.float32),
        ],
    )(seq, wp, bp, wse, bse, wro, bro)


@functools.cache
def _build_k2():
  mesh = plsc.VectorSubcoreMesh(core_axis_name="c", subcore_axis_name="s")
  deco = functools.partial(
    pl.kernel,
    out_type=[
        jax.ShapeDtypeStruct((B, N), jnp.float32),
        jax.ShapeDtypeStruct((B, N), jnp.float32),
        jax.ShapeDtypeStruct((B, N), jnp.float32),
        jax.ShapeDtypeStruct((B, N), jnp.float32),
        jax.ShapeDtypeStruct((B, 2, 16), jnp.float32),
    ],
    mesh=mesh,
    scratch_types=[
        pltpu.VMEM((S * TSE_W,), jnp.float32),
        pltpu.VMEM((CHUNK,), jnp.int32),
        pltpu.VMEM((CHUNK,), jnp.int32),
        pltpu.VMEM((CHUNK,), jnp.float32),
        pltpu.VMEM((CHUNK,), jnp.float32),
        pltpu.VMEM((CHUNK,), jnp.float32),
        pltpu.VMEM((CHUNK,), jnp.float32),
        pltpu.VMEM((CHUNK,), jnp.float32),
        pltpu.VMEM((416,), jnp.float32),
        pltpu.VMEM((16,), jnp.float32),
    ],
    compiler_params=pltpu.CompilerParams(needs_layout_passes=False),
  )

  @deco
  def k2(tse_hbm, st_hbm, en_hbm, mk_hbm, w2_hbm,
        s0_hbm, s1_hbm, s2_hbm, pr_hbm, sq_hbm,
        tab_v, st_v, en_v, mk_v, o0_v, o1_v, o2_v, opr_v, w2_v, sq_v):
    wid = lax.axis_index("s") * 2 + lax.axis_index("c")
    b = wid // 2
    half = wid % 2

    pltpu.sync_copy(tse_hbm.at[b], tab_v)
    pltpu.sync_copy(w2_hbm, w2_v)

    zi = jnp.zeros((16,), jnp.int32)
    b2s = [plsc.load_gather(w2_v, [zi + (384 + l)]) for l in range(3)]

    def chunk_step(ci, seqmax):
        base = half * HALF + ci * CHUNK
        pltpu.sync_copy(st_hbm.at[b, pl.ds(base, CHUNK)], st_v)
        pltpu.sync_copy(en_hbm.at[b, pl.ds(base, CHUNK)], en_v)
        pltpu.sync_copy(mk_hbm.at[b, pl.ds(base, CHUNK)], mk_v)

        def gblock_step(gb, seqmax):
            g0 = gb * (G * 16)
            svs = [st_v[pl.ds(g0 + g * 16, 16)] for g in range(G)]
            evs = [en_v[pl.ds(g0 + g * 16, 16)] for g in range(G)]
            fs0 = tuple(sv * TSE_W for sv in svs)
            fe0 = tuple(ev * TSE_W + HID for ev in evs)
            za = tuple(jnp.zeros((16,), jnp.float32) for _ in range(3 * G))

            def h_step(h, carry):
                accs = list(carry[0])
                fss = list(carry[1])
                fes = list(carry[2])
                w2s = [plsc.load_gather(w2_v, [zi + (l * 128) + h])
                       for l in range(3)]
                for g in range(G):
                    ts = plsc.load_gather(tab_v, [fss[g]])
                    te = plsc.load_gather(tab_v, [fes[g]])
                    gv = jnp.maximum(ts + te, 0.0)
                    accs[3 * g + 0] = accs[3 * g + 0] + gv * w2s[0]
                    accs[3 * g + 1] = accs[3 * g + 1] + gv * w2s[1]
                    accs[3 * g + 2] = accs[3 * g + 2] + gv * w2s[2]
                    fss[g] = fss[g] + 1
                    fes[g] = fes[g] + 1
                return (tuple(accs), tuple(fss), tuple(fes))

            accs, _, _ = lax.fori_loop(0, HID, h_step, (za, fs0, fe0))
            for g in range(G):
                off = g0 + g * 16
                a0 = accs[3 * g + 0] + b2s[0]
                a1 = accs[3 * g + 1] + b2s[1]
                a2 = accs[3 * g + 2] + b2s[2]
                mkv = mk_v[pl.ds(off, 16)]
                prune = jnp.where(mkv > 0.0, jnp.maximum(a1, a2), -1e30)
                o0_v[pl.ds(off, 16)] = a0
                o1_v[pl.ds(off, 16)] = a1
                o2_v[pl.ds(off, 16)] = a2
                opr_v[pl.ds(off, 16)] = prune
                ef = evs[g].astype(jnp.float32)
                seqmax = jnp.maximum(seqmax, mkv * ef)
            return seqmax

        seqmax = lax.fori_loop(0, CHUNK // (G * 16), gblock_step, seqmax)
        pltpu.sync_copy(o0_v, s0_hbm.at[b, pl.ds(base, CHUNK)])
        pltpu.sync_copy(o1_v, s1_hbm.at[b, pl.ds(base, CHUNK)])
        pltpu.sync_copy(o2_v, s2_hbm.at[b, pl.ds(base, CHUNK)])
        pltpu.sync_copy(opr_v, pr_hbm.at[b, pl.ds(base, CHUNK)])
        return seqmax

    seqmax = lax.fori_loop(0, HALF // CHUNK, chunk_step,
                           jnp.zeros((16,), jnp.float32))
    sq_v[...] = seqmax
    pltpu.sync_copy(sq_v, sq_hbm.at[b, half])

  return k2


def _k3_body(prm_ref, pk_ref, cp_ref, ci_ref):
    x = prm_ref[...]
    pk = pk_ref[...]
    fio = (lax.broadcasted_iota(jnp.int32, (B, 128, 128), 1) * 128
           + lax.broadcasted_iota(jnp.int32, (B, 128, 128), 2))
    ic = lax.broadcasted_iota(jnp.int32, (B, CAND), 1)
    bigi = jnp.int32(2147483647)

    def step(k, carry):
        x, cp, ci = carry
        m = jnp.max(jnp.max(x, axis=2, keepdims=True), axis=1, keepdims=True)
        flat = jnp.min(jnp.min(jnp.where(x == m, fio, bigi), axis=2,
                               keepdims=True), axis=1, keepdims=True)
        hit = fio == flat
        pv = jnp.max(jnp.max(jnp.where(hit, pk, 0), axis=2, keepdims=True),
                     axis=1, keepdims=True)
        sel = ic == k
        cp = jnp.where(sel, jnp.broadcast_to(pv[:, :, 0], (B, CAND)), cp)
        ci = jnp.where(sel, jnp.broadcast_to(flat[:, :, 0], (B, CAND)), ci)
        x = jnp.where(hit, -3e38, x)
        return x, cp, ci

    zi = jnp.zeros((B, CAND), jnp.int32)
    x, cp, ci = lax.fori_loop(0, CAND, step, (x, zi, zi))
    cp_ref[...] = cp
    ci_ref[...] = ci


def _k3(prm, pk):
    return pl.pallas_call(
        _k3_body,
        in_specs=[
            pl.BlockSpec((B, 128, 128), lambda: (0, 0, 0)),
            pl.BlockSpec((B, 128, 128), lambda: (0, 0, 0)),
        ],
        out_specs=[
            pl.BlockSpec((B, CAND), lambda: (0, 0)),
            pl.BlockSpec((B, CAND), lambda: (0, 0)),
        ],
        out_shape=[
            jax.ShapeDtypeStruct((B, CAND), jnp.int32),
            jax.ShapeDtypeStruct((B, CAND), jnp.int32),
        ],
    )(prm, pk)


def _k4_body(cp_ref, ci_ref, h_ref, w1_ref, b1_ref, w2_ref,
             b2_ref, tro_ref, sq_ref, r2_ref, rb2_ref,
             role_ref, tops_ref, tope_ref, tsm_ref):
    cpv = cp_ref[0]
    csv = lax.shift_right_logical(cpv, 10)
    cev = lax.shift_right_logical(cpv, 1) & 511
    cmv = (cpv & 1).astype(jnp.float32)
    civ = ci_ref[0]
    i512c = lax.broadcasted_iota(jnp.int32, (CAND, S), 1)
    ohs = jnp.where(csv.reshape(CAND, 1) == i512c, 1.0, 0.0)
    ohe = jnp.where(cev.reshape(CAND, 1) == i512c, 1.0, 0.0)
    embs = jnp.dot(ohs, h_ref[0], preferred_element_type=jnp.float32,
                   precision=lax.Precision.HIGHEST)
    embe = jnp.dot(ohe, h_ref[0], preferred_element_type=jnp.float32,
                   precision=lax.Precision.HIGHEST)
    emb = jnp.concatenate([embs, embe], axis=1)
    u = jnp.dot(emb, w1_ref[...], preferred_element_type=jnp.float32) + b1_ref[...]
    g = jnp.maximum(u, 0.0)
    sc3 = jnp.dot(g, w2_ref[...], preferred_element_type=jnp.float32) + b2_ref[...]
    pr = jnp.max(sc3[:, 1:3], axis=1, keepdims=True)
    cmc = cmv.reshape(CAND, 1)
    pexc = jnp.where(cmc > 0.0, pr, -1e30)
    civc = civ.reshape(CAND, 1)
    csvc = csv.reshape(CAND, 1)
    cevc = cev.reshape(CAND, 1)

    pexr = pexc.reshape(1, CAND)
    civr = civc.reshape(1, CAND)
    better = (pexc > pexr) | ((pexc == pexr) & (civc < civr))
    rank = jnp.sum(better.astype(jnp.int32), axis=0, keepdims=True)
    rankc = rank.reshape(CAND, 1)

    i64 = lax.broadcasted_iota(jnp.int32, (1, K), 1)
    hitjk = rankc == i64
    tops = jnp.max(jnp.where(hitjk, csvc, 0), axis=0, keepdims=True)
    tope = jnp.max(jnp.where(hitjk, cevc, 0), axis=0, keepdims=True)
    topm = jnp.max(jnp.where(hitjk, cmc, 0.0), axis=0, keepdims=True)

    sl = jnp.max(sq_ref[...])
    nk = jnp.clip(jnp.ceil(SPW * sl), 1.0, float(K))
    tsm = jnp.where(i64.astype(jnp.float32) < nk, topm, 0.0)

    i512 = lax.broadcasted_iota(jnp.int32, (K, S), 1)
    ohs2 = jnp.where(tops.reshape(K, 1) == i512, 1.0, 0.0)
    ohe2 = jnp.where(tope.reshape(K, 1) == i512, 1.0, 0.0)
    gs = jnp.dot(ohs2, tro_ref[0], preferred_element_type=jnp.float32)
    ge = jnp.dot(ohe2, tro_ref[0], preferred_element_type=jnp.float32)
    p = gs[:, 0:HID] + ge[:, HID:2 * HID]
    q = gs[:, 2 * HID:3 * HID] + ge[:, 3 * HID:4 * HID]
    hid = jnp.maximum(
        jnp.broadcast_to(p.reshape(K, 1, HID), (K, K, HID))
        + jnp.broadcast_to(q.reshape(1, K, HID), (K, K, HID)), 0.0
    ).reshape(K * K, HID)
    role_ref[0] = jnp.dot(hid, r2_ref[...], preferred_element_type=jnp.float32) + rb2_ref[...]
    tops_ref[0] = tops
    tope_ref[0] = tope
    tsm_ref[0] = tsm


def _k4(cp, ci, h, w1, b1, w2, b2, tro, sq, r2, rb2):
    return pl.pallas_call(
        _k4_body,
        grid=(B,),
        in_specs=[
            pl.BlockSpec((1, 1, CAND), lambda b: (b, 0, 0)),
            pl.BlockSpec((1, 1, CAND), lambda b: (b, 0, 0)),
            pl.BlockSpec((1, S, H), lambda b: (b, 0, 0)),
            pl.BlockSpec((2 * H, HID), lambda b: (0, 0)),
            pl.BlockSpec((1, HID), lambda b: (0, 0)),
            pl.BlockSpec((HID, 3), lambda b: (0, 0)),
            pl.BlockSpec((1, 3), lambda b: (0, 0)),
            pl.BlockSpec((1, S, TROLE_W), lambda b: (b, 0, 0)),
            pl.BlockSpec((1, 1, 32), lambda b: (b, 0, 0)),
            pl.BlockSpec((HID, 2), lambda b: (0, 0)),
            pl.BlockSpec((1, 2), lambda b: (0, 0)),
        ],
        out_specs=[
            pl.BlockSpec((1, K * K, 2), lambda b: (b, 0, 0)),
            pl.BlockSpec((1, 1, K), lambda b: (b, 0, 0)),
            pl.BlockSpec((1, 1, K), lambda b: (b, 0, 0)),
            pl.BlockSpec((1, 1, K), lambda b: (b, 0, 0)),
        ],
        out_shape=[
            jax.ShapeDtypeStruct((B, K * K, 2), jnp.float32),
            jax.ShapeDtypeStruct((B, 1, K), jnp.int32),
            jax.ShapeDtypeStruct((B, 1, K), jnp.int32),
            jax.ShapeDtypeStruct((B, 1, K), jnp.float32),
        ],
    )(cp, ci, h, w1, b1, w2, b2, tro, sq, r2, rb2)


def kernel(seq_tensor, span_indices, span_mask, W_proj, b_proj, W1, b1, W2,
           b2, R1, rb1, R2, rb2):
    f32 = jnp.float32
    starts = span_indices[..., 0]
    ends = span_indices[..., 1]

    wse = jnp.concatenate([W1[:H], W1[H:]], axis=1)
    bse = jnp.concatenate([jnp.zeros((HID,), f32), b1])[None, :]
    wro = jnp.concatenate([R1[:H], R1[H:2 * H], R1[2 * H:3 * H], R1[3 * H:]],
                          axis=1)
    bro = jnp.concatenate([jnp.zeros((HID,), f32), rb1,
                           jnp.zeros((2 * HID,), f32)])[None, :]
    pad28 = jnp.zeros((28,), f32)
    w2cat = jnp.concatenate([W2[:, 0], pad28, W2[:, 1], pad28, W2[:, 2],
                             pad28, b2, jnp.zeros((29,), f32)])

    tse, tro, h3 = _k1(seq_tensor, W_proj, b_proj[None, :], wse, bse, wro, bro)

    s0, s1, s2, prm, seqpart = _build_k2()(
        tse.reshape(B, S * TSE_W), starts, ends, span_mask, w2cat)

    pk = (starts * 1024 + ends * 2
          + (span_mask > 0).astype(jnp.int32)).reshape(B, 128, 128)
    cp, ci = _k3(prm.reshape(B, 128, 128), pk)
    cp = cp.reshape(B, 1, CAND)
    ci = ci.reshape(B, 1, CAND)

    role, tops, tope, tsm = _k4(
        cp, ci, h3, W1, b1[None, :], W2, b2[None, :], tro,
        seqpart.reshape(B, 1, 32), R2, rb2[None, :])

    span_scores = jnp.stack([s0, s1, s2], axis=-1)
    role_scores = role.reshape(B, K, K, 2)
    top_indices = jnp.stack([tops.reshape(B, K), tope.reshape(B, K)], axis=-1)
    return span_scores, role_scores, tsm.reshape(B, K), top_indices
